# baseline (speedup 1.0000x reference)
import functools

import jax
import jax.numpy as jnp
from jax import lax
from jax.experimental import pallas as pl
from jax.experimental.pallas import tpu as pltpu
from jax.experimental.pallas import tpu_sc as plsc

_TN = 256
_MASK = -2048
_TOP = 2147483647
_L = 16
_CH = 128


def _nn_body(u_ref, k_ref, idx_ref, w_ref, *, tpb, m):
    t = pl.program_id(0)
    u = u_ref[0]
    kk = k_ref[0]
    dt = jnp.dot(kk, u, preferred_element_type=jnp.float32)
    u2 = (u[0:1] * u[0:1] + u[1:2] * u[1:2]) + u[2:3] * u[2:3]
    k2 = (kk[:, 0:1] * kk[:, 0:1] + kk[:, 1:2] * kk[:, 1:2]) + kk[:, 2:3] * kk[:, 2:3]
    d2 = (u2 - 2.0 * dt) + k2
    ib = jax.lax.bitcast_convert_type(d2, jnp.int32)
    mk = ib ^ (jnp.right_shift(ib, 31) & jnp.int32(_TOP))
    big = jnp.int32(_TOP)
    cand = jax.lax.broadcasted_iota(jnp.int32, d2.shape, 0)
    m1 = jnp.min(mk, axis=0, keepdims=True)
    kx1 = jnp.where(mk <= m1, big, mk)
    m2 = jnp.min(kx1, axis=0, keepdims=True)
    kx2 = jnp.where(kx1 <= m2, big, kx1)
    m3 = jnp.min(kx2, axis=0, keepdims=True)
    i1 = jnp.min(jnp.where(mk <= m1, cand, big), axis=0, keepdims=True)
    i2 = jnp.min(jnp.where(kx1 <= m2, cand, big), axis=0, keepdims=True)
    i3 = jnp.min(jnp.where(kx2 <= m3, cand, big), axis=0, keepdims=True)

    tod = lambda v: jax.lax.bitcast_convert_type(
        v ^ (jnp.right_shift(v, 31) & jnp.int32(_TOP)), jnp.float32)
    r1 = 1.0 / (tod(m1) + 1e-8)
    r2 = 1.0 / (tod(m2) + 1e-8)
    r3 = 1.0 / (tod(m3) + 1e-8)
    nrm = (r1 + r2) + r3
    base = (t // tpb) * m
    idx_ref[...] = jnp.concatenate([i1 + base, i2 + base, i3 + base], axis=0)
    w_ref[...] = jnp.concatenate([r1 / nrm, r2 / nrm, r3 / nrm], axis=0)


def _three_nn_pallas(uT8, k8, tn):
    B, _, N = uT8.shape
    M = k8.shape[1]
    tpb = N // tn
    nt = B * tpb
    return pl.pallas_call(
        functools.partial(_nn_body, tpb=tpb, m=M),
        grid=(nt,),
        in_specs=[
            pl.BlockSpec((1, 8, tn), lambda t: (t // tpb, 0, t % tpb)),
            pl.BlockSpec((1, M, 8), lambda t: (t // tpb, 0, 0)),
        ],
        out_specs=[
            pl.BlockSpec((3, tn), lambda t: (0, t)),
            pl.BlockSpec((3, tn), lambda t: (0, t)),
        ],
        out_shape=[
            jax.ShapeDtypeStruct((3, B * N), jnp.int32),
            jax.ShapeDtypeStruct((3, B * N), jnp.float32),
        ],
        compiler_params=pltpu.CompilerParams(
            dimension_semantics=("arbitrary",),
        ),
    )(uT8, k8)


def _sc_body(idx_hbm, w_hbm, tab_hbm, out_hbm,
             i0v, i1v, i2v, w0v, w1v, w2v, r0v, r1v, r2v, accv, sem,
             *, npw, c2):
    nc = 2
    wid = lax.axis_index("s") * nc + lax.axis_index("c")
    base = wid * npw
    nch = npw // _CH

    def chunk(ci, carry):
        off = base + ci * _CH
        bn = npw * 32
        pltpu.sync_copy(idx_hbm.at[pl.ds(off, _CH)], i0v)
        pltpu.sync_copy(idx_hbm.at[pl.ds(bn + off, _CH)], i1v)
        pltpu.sync_copy(idx_hbm.at[pl.ds(2 * bn + off, _CH)], i2v)
        pltpu.sync_copy(w_hbm.at[pl.ds(off, _CH)], w0v)
        pltpu.sync_copy(w_hbm.at[pl.ds(bn + off, _CH)], w1v)
        pltpu.sync_copy(w_hbm.at[pl.ds(2 * bn + off, _CH)], w2v)
        c0 = pltpu.async_copy(tab_hbm.at[i0v], r0v, sem)
        c1 = pltpu.async_copy(tab_hbm.at[i1v], r1v, sem)
        c2_ = pltpu.async_copy(tab_hbm.at[i2v], r2v, sem)
        c0.wait()
        c1.wait()
        c2_.wait()

        def p16_loop(g, carry2):
            w16_0 = w0v[pl.ds(g * _L, _L)]
            w16_1 = w1v[pl.ds(g * _L, _L)]
            w16_2 = w2v[pl.ds(g * _L, _L)]
            dn = lax.GatherDimensionNumbers(
                offset_dims=(), collapsed_slice_dims=(0,), start_index_map=(0,))
            for pi in range(_L):
                p = g * _L + pi
                iv = jnp.full((_L, 1), pi, jnp.int32)
                gat = lambda v: lax.gather(
                    v, iv, dn, (1,),
                    mode=lax.GatherScatterMode.PROMISE_IN_BOUNDS)
                s0 = gat(w16_0)
                s1 = gat(w16_1)
                s2 = gat(w16_2)
                for j in range(c2 // _L):
                    sl = pl.ds(j * _L, _L)
                    accv[p, sl] = (s0 * r0v[p, sl] + s1 * r1v[p, sl]
                                   + s2 * r2v[p, sl])
            return carry2

        lax.fori_loop(0, _CH // _L, p16_loop, 0, unroll=False)
        pltpu.sync_copy(accv, out_hbm.at[pl.ds(off, _CH)])
        return carry

    lax.fori_loop(0, nch, chunk, 0, unroll=False)


def _interp_sc(idx, w, table):
    BM, C2 = table.shape
    BN = idx.shape[1]
    idx = idx.reshape(3 * BN)
    w = w.reshape(3 * BN)
    nw = 32
    npw = BN // nw
    mesh = plsc.VectorSubcoreMesh(core_axis_name="c", subcore_axis_name="s")
    f = pl.kernel(
        functools.partial(_sc_body, npw=npw, c2=C2),
        mesh=mesh,
        out_type=jax.ShapeDtypeStruct((BN, C2), jnp.float32),
        scratch_types=[
            pltpu.VMEM((_CH,), jnp.int32),
            pltpu.VMEM((_CH,), jnp.int32),
            pltpu.VMEM((_CH,), jnp.int32),
            pltpu.VMEM((_CH,), jnp.float32),
            pltpu.VMEM((_CH,), jnp.float32),
            pltpu.VMEM((_CH,), jnp.float32),
            pltpu.VMEM((_CH, C2), jnp.float32),
            pltpu.VMEM((_CH, C2), jnp.float32),
            pltpu.VMEM((_CH, C2), jnp.float32),
            pltpu.VMEM((_CH, C2), jnp.float32),
            pltpu.SemaphoreType.DMA,
        ],
    )
    return f(idx, w, table)


def _mlp_body(it_ref, uf_ref, w0a_ref, w0b_ref, w1_ref,
              b0_ref, g0_ref, bt0_ref, b1_ref, g1_ref, bt1_ref,
              out_ref, y0_scr, y1_scr, st_scr, *, bn):
    p = pl.program_id(0)
    t = pl.program_id(1)
    ninv = 1.0 / float(bn)

    @pl.when(p == 0)
    def _phase0():
        interp = it_ref[...].T
        y0 = (jnp.dot(w0a_ref[...], interp, preferred_element_type=jnp.float32)
              + jnp.dot(w0b_ref[...], uf_ref[0], preferred_element_type=jnp.float32)
              + b0_ref[...])
        y0_scr[t] = y0

        @pl.when(t == 0)
        def _init():
            st_scr[...] = jnp.zeros_like(st_scr)

        st_scr[:, 0:1] += jnp.sum(y0, axis=1, keepdims=True)
        st_scr[:, 1:2] += jnp.sum(y0 * y0, axis=1, keepdims=True)

    @pl.when(p == 1)
    def _phase1():
        y0 = y0_scr[t]
        mean = st_scr[:, 0:1] * ninv
        var = st_scr[:, 1:2] * ninv - mean * mean
        sc = g0_ref[...] * jax.lax.rsqrt(var + 1e-5)
        sh = bt0_ref[...] - mean * sc
        h = jnp.maximum(y0 * sc + sh, 0.0)
        y1 = jnp.dot(w1_ref[...], h, preferred_element_type=jnp.float32) + b1_ref[...]
        y1_scr[t] = y1

        @pl.when(t == 0)
        def _init():
            st_scr[:, 2:4] = jnp.zeros_like(st_scr[:, 2:4])

        st_scr[:, 2:3] += jnp.sum(y1, axis=1, keepdims=True)
        st_scr[:, 3:4] += jnp.sum(y1 * y1, axis=1, keepdims=True)

    @pl.when(p == 2)
    def _phase2():
        y1 = y1_scr[t]
        mean = st_scr[:, 2:3] * ninv
        var = st_scr[:, 3:4] * ninv - mean * mean
        sc = g1_ref[...] * jax.lax.rsqrt(var + 1e-5)
        sh = bt1_ref[...] - mean * sc
        out_ref[0] = jnp.maximum(y1 * sc + sh, 0.0)


def _mlp_pallas(interp, uf, w0a, w0b, W1, vecs, tn):
    B, C1, N = uf.shape
    K0 = w0a.shape[0]
    K1 = W1.shape[0]
    tpb = N // tn
    nt = B * tpb
    bn = B * N

    def off(p, t):
        return jnp.where(p == 0, t // tpb, 0), 0, jnp.where(p == 0, t % tpb, 0)

    return pl.pallas_call(
        functools.partial(_mlp_body, bn=bn),
        grid=(3, nt),
        in_specs=[
            pl.BlockSpec((tn, K0), lambda p, t: (jnp.where(p == 0, t, 0), 0)),
            pl.BlockSpec((1, C1, tn), off),
            pl.BlockSpec((K0, w0a.shape[1]), lambda p, t: (0, 0)),
            pl.BlockSpec((K0, C1), lambda p, t: (0, 0)),
            pl.BlockSpec((K1, K0), lambda p, t: (0, 0)),
        ] + [pl.BlockSpec((K0, 1), lambda p, t: (0, 0))] * 3
          + [pl.BlockSpec((K1, 1), lambda p, t: (0, 0))] * 3,
        out_specs=pl.BlockSpec(
            (1, K1, tn),
            lambda p, t: (jnp.where(p == 2, t // tpb, 0), 0,
                          jnp.where(p == 2, t % tpb, 0))),
        out_shape=jax.ShapeDtypeStruct((B, K1, N), jnp.float32),
        scratch_shapes=[
            pltpu.VMEM((nt, K0, tn), jnp.float32),
            pltpu.VMEM((nt, K1, tn), jnp.float32),
            pltpu.VMEM((K0, 8), jnp.float32),
        ],
        compiler_params=pltpu.CompilerParams(
            dimension_semantics=("arbitrary", "arbitrary"),
            vmem_limit_bytes=100 * 1024 * 1024,
        ),
    )(interp, uf, w0a, w0b, W1, *vecs)


@jax.jit
def kernel(unknown, known, unknow_feats, known_feats,
           W0, b0, gamma0, beta0, W1, b1, gamma1, beta1):
    B, N, _ = unknown.shape
    M = known.shape[1]
    C2 = known_feats.shape[1]
    tn = _TN

    uT8 = jnp.concatenate(
        [jnp.swapaxes(unknown, 1, 2),
         jnp.zeros((B, 5, N), unknown.dtype)], axis=1)
    k8 = jnp.concatenate(
        [known, jnp.zeros((B, M, 5), known.dtype)], axis=2)

    idx, w = _three_nn_pallas(uT8, k8, tn)
    table = jnp.swapaxes(known_feats, 1, 2).reshape(B * M, C2)
    interp = _interp_sc(idx, w, table)

    col = lambda v: v.reshape(-1, 1)
    vecs = (col(b0), col(gamma0), col(beta0), col(b1), col(gamma1), col(beta1))
    return _mlp_pallas(interp, unknow_feats, W0[:, :C2], W0[:, C2:], W1,
                       vecs, tn)
